# baseline (speedup 1.0000x reference)
import functools

import jax
import jax.numpy as jnp
from jax import lax
from jax.experimental import pallas as pl
from jax.experimental.pallas import tpu as pltpu
from jax.experimental.pallas import tpu_sc as plsc

N_NODES = 50000
N_EDGES = 800000
N_OPS = 120
H = 64
G = 512

BK = 512
NP = 50176
NB = NP // BK

NSC = 16
CK = 128
NCH = 392
EP = NSC * CK * NCH
NCHT = NSC * NCH
RPS = NP // NSC
NCH_H = NCHT // 2 // NSC

_SC_MESH = dict(core_axis_name="c", subcore_axis_name="s")
_SC_PARAMS = pltpu.CompilerParams(use_tc_tiling_on_sc=False)


def _sc_aggregate(y0, y1, srcp, dstp, zeros32):
    @functools.partial(
        pl.kernel,
        out_type=jax.ShapeDtypeStruct((2, NP, 32), jnp.float32),
        mesh=plsc.VectorSubcoreMesh(**_SC_MESH),
        scratch_types=[
            pltpu.VMEM((CK,), jnp.int32),
            pltpu.VMEM((CK,), jnp.int32),
            pltpu.VMEM((CK,), jnp.int32),
            pltpu.VMEM((CK,), jnp.int32),
            pltpu.VMEM((CK, 32), jnp.float32),
            pltpu.VMEM((CK, 32), jnp.float32),
            pltpu.VMEM_SHARED((NP, 32), jnp.float32),
            pltpu.SemaphoreType.DMA,
            pltpu.SemaphoreType.DMA,
        ],
        compiler_params=_SC_PARAMS,
    )
    def agg_kernel(y0_hbm, y1_hbm, src_hbm, dst_hbm, zero_hbm, out_hbm,
                   sa, da, sb, db, rows_a, rows_b, acc_sh, sem_a, sem_b):
        c = lax.axis_index("c")
        s = lax.axis_index("s")
        pltpu.sync_copy(zero_hbm, acc_sh.at[pl.ds(s * RPS, RPS)])
        plsc.subcore_barrier()
        base = s * NCH

        def run(tbl):
            pltpu.sync_copy(src_hbm.at[base], sa)
            pltpu.sync_copy(dst_hbm.at[base], da)
            pltpu.async_copy(tbl.at[sa], rows_a, sem_a)

            def step(g, si_x, di_x, rows_x, sem_x, si_y, di_y, rows_y,
                     sem_y, pre):
                pltpu.sync_copy(src_hbm.at[base + g + 1], si_y)
                pltpu.sync_copy(dst_hbm.at[base + g + 1], di_y)
                pltpu.async_copy(tbl.at[si_y], rows_y, sem_y)
                pltpu.make_async_copy(tbl.at[si_x], rows_x, sem_x).wait()
                pltpu.sync_copy(rows_x, acc_sh.at[di_x], add=True)
                if pre:
                    pltpu.sync_copy(src_hbm.at[base + g + 2], si_x)
                    pltpu.sync_copy(dst_hbm.at[base + g + 2], di_x)
                    pltpu.async_copy(tbl.at[si_x], rows_x, sem_x)

            @pl.loop(0, NCH - 2, step=2)
            def _(g):
                step(g, sa, da, rows_a, sem_a, sb, db, rows_b, sem_b, True)
                pltpu.make_async_copy(tbl.at[sb], rows_b, sem_b).wait()
                pltpu.sync_copy(rows_b, acc_sh.at[db], add=True)

            step(NCH - 2, sa, da, rows_a, sem_a, sb, db, rows_b, sem_b,
                 False)
            pltpu.make_async_copy(tbl.at[sb], rows_b, sem_b).wait()
            pltpu.sync_copy(rows_b, acc_sh.at[db], add=True)

        @pl.when(c == 0)
        def _():
            run(y0_hbm)

        @pl.when(c == 1)
        def _():
            run(y1_hbm)

        plsc.subcore_barrier()
        pltpu.sync_copy(acc_sh.at[pl.ds(s * RPS, RPS)],
                        out_hbm.at[c].at[pl.ds(s * RPS, RPS)])

    return agg_kernel(y0, y1, srcp, dstp, zeros32)


def _sc_counts(dstp, zeros16, ones16):
    @functools.partial(
        pl.kernel,
        out_type=jax.ShapeDtypeStruct((2, NP, 16), jnp.float32),
        mesh=plsc.VectorSubcoreMesh(**_SC_MESH),
        scratch_types=[
            pltpu.VMEM((CK,), jnp.int32),
            pltpu.VMEM((CK, 16), jnp.float32),
            pltpu.VMEM_SHARED((NP, 16), jnp.float32),
        ],
        compiler_params=_SC_PARAMS,
    )
    def cnt_kernel(dst_hbm, zero_hbm, ones_hbm, out_hbm,
                   idx_v, ones_v, acc_sh):
        c = lax.axis_index("c")
        s = lax.axis_index("s")
        pltpu.sync_copy(zero_hbm, acc_sh.at[pl.ds(s * RPS, RPS)])
        pltpu.sync_copy(ones_hbm, ones_v)
        plsc.subcore_barrier()
        base = (c * NSC + s) * NCH_H

        @pl.loop(0, NCH_H)
        def _(gi):
            pltpu.sync_copy(dst_hbm.at[base + gi], idx_v)
            pltpu.sync_copy(ones_v, acc_sh.at[idx_v], add=True)

        plsc.subcore_barrier()
        pltpu.sync_copy(acc_sh.at[pl.ds(s * RPS, RPS)],
                        out_hbm.at[c].at[pl.ds(s * RPS, RPS)])

    return cnt_kernel(dstp, zeros16, ones16)


def _inv_counts(cnt):
    def body(c_r, o_r):
        cn = c_r[0, :, 0:1] + c_r[1, :, 0:1]
        o_r[...] = jnp.broadcast_to(1.0 / jnp.maximum(cn, 1.0), (BK, 8))

    return pl.pallas_call(
        body,
        grid=(NB,),
        in_specs=[pl.BlockSpec((2, BK, 16), lambda i: (0, i, 0))],
        out_specs=pl.BlockSpec((BK, 8), lambda i: (i, 0)),
        out_shape=jax.ShapeDtypeStruct((NP, 8), jnp.float32),
    )(cnt)


def _input_layer0(node_feat, node_config, opcode3d, op_emb, wa, wmid, wc, b,
                  pw, pb, lw, rw, lb):
    def body(nf, cf, opc, emb, wa_r, wm_r, wc_r, b_r, pw_r, pb_r, lw_r,
             rw_r, lb_r, y0_r, y1_r, z_r):
        ot = jnp.dot(emb[...], wm_r[...], preferred_element_type=jnp.float32)
        op = opc[...]
        oh = (op == lax.broadcasted_iota(jnp.int32, (BK, N_OPS), 1)
              ).astype(jnp.float32)
        acc = jnp.dot(nf[...], wa_r[...], preferred_element_type=jnp.float32)
        acc += jnp.dot(oh, ot, preferred_element_type=jnp.float32)
        acc += jnp.dot(cf[...], wc_r[...], preferred_element_type=jnp.float32)
        acc += b_r[...]
        x = jnp.maximum(acc, 0.0)
        h = jnp.maximum(
            jnp.dot(x, pw_r[...], preferred_element_type=jnp.float32)
            + pb_r[...], 0.0)
        y = jnp.dot(h, lw_r[...], preferred_element_type=jnp.float32)
        y0_r[...] = y[:, :32]
        y1_r[...] = y[:, 32:]
        z_r[...] = (jnp.dot(h, rw_r[...], preferred_element_type=jnp.float32)
                    + lb_r[...])

    return pl.pallas_call(
        body,
        grid=(NB,),
        in_specs=[
            pl.BlockSpec((BK, 140), lambda i: (i, 0)),
            pl.BlockSpec((BK, 18), lambda i: (i, 0)),
            pl.BlockSpec((BK, 1), lambda i: (i, 0)),
            pl.BlockSpec((N_OPS, 32), lambda i: (0, 0)),
            pl.BlockSpec((140, 128), lambda i: (0, 0)),
            pl.BlockSpec((32, 128), lambda i: (0, 0)),
            pl.BlockSpec((18, 128), lambda i: (0, 0)),
            pl.BlockSpec((1, 128), lambda i: (0, 0)),
            pl.BlockSpec((128, 128), lambda i: (0, 0)),
            pl.BlockSpec((1, 128), lambda i: (0, 0)),
            pl.BlockSpec((128, H), lambda i: (0, 0)),
            pl.BlockSpec((128, H), lambda i: (0, 0)),
            pl.BlockSpec((1, H), lambda i: (0, 0)),
        ],
        out_specs=[
            pl.BlockSpec((BK, 32), lambda i: (i, 0)),
            pl.BlockSpec((BK, 32), lambda i: (i, 0)),
            pl.BlockSpec((BK, H), lambda i: (i, 0)),
        ],
        out_shape=[
            jax.ShapeDtypeStruct((NP, 32), jnp.float32),
            jax.ShapeDtypeStruct((NP, 32), jnp.float32),
            jax.ShapeDtypeStruct((NP, H), jnp.float32),
        ],
    )(node_feat, node_config, opcode3d, op_emb, wa, wmid, wc, b,
      pw, pb, lw, rw, lb)


def _combine(agg, inv, z):
    def body(a_r, i_r, z_r, o_r):
        s = jnp.concatenate([a_r[0], a_r[1]], axis=1)
        x = s * i_r[:, 0:1] + z_r[...]
        nrm = jnp.sqrt(jnp.sum(x * x, axis=1, keepdims=True))
        o_r[...] = x / jnp.maximum(nrm, 1e-12)

    return pl.pallas_call(
        body,
        grid=(NB,),
        in_specs=[
            pl.BlockSpec((2, BK, 32), lambda i: (0, i, 0)),
            pl.BlockSpec((BK, 8), lambda i: (i, 0)),
            pl.BlockSpec((BK, H), lambda i: (i, 0)),
        ],
        out_specs=pl.BlockSpec((BK, H), lambda i: (i, 0)),
        out_shape=jax.ShapeDtypeStruct((NP, H), jnp.float32),
    )(agg, inv, z)


def _layer_matmuls(x, pw, pb, lw, rw, lb):
    d = x.shape[1]

    def body(x_r, pw_r, pb_r, lw_r, rw_r, lb_r, y0_r, y1_r, z_r):
        h = jnp.maximum(
            jnp.dot(x_r[...], pw_r[...], preferred_element_type=jnp.float32)
            + pb_r[...], 0.0)
        y = jnp.dot(h, lw_r[...], preferred_element_type=jnp.float32)
        y0_r[...] = y[:, :32]
        y1_r[...] = y[:, 32:]
        z_r[...] = (jnp.dot(h, rw_r[...], preferred_element_type=jnp.float32)
                    + lb_r[...])

    return pl.pallas_call(
        body,
        grid=(NB,),
        in_specs=[
            pl.BlockSpec((BK, d), lambda i: (i, 0)),
            pl.BlockSpec((d, d), lambda i: (0, 0)),
            pl.BlockSpec((1, d), lambda i: (0, 0)),
            pl.BlockSpec((d, H), lambda i: (0, 0)),
            pl.BlockSpec((d, H), lambda i: (0, 0)),
            pl.BlockSpec((1, H), lambda i: (0, 0)),
        ],
        out_specs=[
            pl.BlockSpec((BK, 32), lambda i: (i, 0)),
            pl.BlockSpec((BK, 32), lambda i: (i, 0)),
            pl.BlockSpec((BK, H), lambda i: (i, 0)),
        ],
        out_shape=[
            jax.ShapeDtypeStruct((NP, 32), jnp.float32),
            jax.ShapeDtypeStruct((NP, 32), jnp.float32),
            jax.ShapeDtypeStruct((NP, H), jnp.float32),
        ],
    )(x, pw, pb, lw, rw, lb)


def _pool_combine(agg, inv, z, batch3d, batchcol, post_w, post_b):
    def body(a_r, i_r, z_r, b_r, bc_r, pw_r, pb_r, o_r, smax, ssum, scnt):
        i = pl.program_id(0)

        @pl.when(i == 0)
        def _():
            smax[...] = jnp.full((G, H), -jnp.inf, jnp.float32)
            ssum[...] = jnp.zeros((G, H), jnp.float32)
            scnt[...] = jnp.zeros((G, 8), jnp.float32)

        s = jnp.concatenate([a_r[0], a_r[1]], axis=1)
        x = s * i_r[:, 0:1] + z_r[...]
        nrm = jnp.sqrt(jnp.sum(x * x, axis=1, keepdims=True))
        x = x / jnp.maximum(nrm, 1e-12)

        nrow = i * BK + lax.broadcasted_iota(jnp.int32, (1, BK), 1)
        bm = jnp.where(nrow < N_NODES, b_r[0], -1)
        onehot = (bm == lax.broadcasted_iota(jnp.int32, (G, BK), 0)
                  ).astype(jnp.float32)
        ssum[...] += jnp.dot(onehot, x, preferred_element_type=jnp.float32)
        scnt[...] += jnp.dot(onehot, jnp.ones((BK, 8), jnp.float32),
                             preferred_element_type=jnp.float32)

        g_lo = jnp.min(jnp.where(nrow < N_NODES, bm, G))
        g_hi = jnp.max(bm)

        ncol = i * BK + lax.broadcasted_iota(jnp.int32, (BK, 1), 0)
        bmc = jnp.where(ncol < N_NODES, bc_r[...], -1)

        def upd(g, carry):
            col = jnp.max(jnp.where(bmc == g, x, -jnp.inf),
                          axis=0, keepdims=True)
            smax[pl.ds(g, 1), :] = jnp.maximum(smax[pl.ds(g, 1), :], col)
            return carry

        lax.fori_loop(g_lo, g_hi + 1, upd, 0)

        @pl.when(i == NB - 1)
        def _():
            mean = ssum[...] / jnp.maximum(scnt[...][:, 0:1], 1.0)
            mx = smax[...]
            mx = jnp.where(mx == -jnp.inf, 0.0, mx)
            gv = mx + mean
            gv = gv / jnp.sqrt(jnp.sum(gv * gv, axis=1, keepdims=True))
            o_r[...] = (jnp.dot(gv, pw_r[...],
                                preferred_element_type=jnp.float32)
                        + pb_r[...])

    return pl.pallas_call(
        body,
        grid=(NB,),
        in_specs=[
            pl.BlockSpec((2, BK, 32), lambda i: (0, i, 0)),
            pl.BlockSpec((BK, 8), lambda i: (i, 0)),
            pl.BlockSpec((BK, H), lambda i: (i, 0)),
            pl.BlockSpec((1, 1, BK), lambda i: (i, 0, 0)),
            pl.BlockSpec((BK, 1), lambda i: (i, 0)),
            pl.BlockSpec((H, 1), lambda i: (0, 0)),
            pl.BlockSpec((1, 1), lambda i: (0, 0)),
        ],
        out_specs=pl.BlockSpec((G, 1), lambda i: (0, 0)),
        out_shape=jax.ShapeDtypeStruct((G, 1), jnp.float32),
        scratch_shapes=[
            pltpu.VMEM((G, H), jnp.float32),
            pltpu.VMEM((G, H), jnp.float32),
            pltpu.VMEM((G, 8), jnp.float32),
        ],
    )(agg, inv, z, batch3d, batchcol, post_w, post_b)


def kernel(node_feat, node_config_feat, node_opcode, edge_index, batch, op_emb,
           lin_W, lin_b,
           proj_W0, proj_b0, lin_l_W0, lin_l_b0, lin_r_W0,
           proj_W1, proj_b1, lin_l_W1, lin_l_b1, lin_r_W1,
           proj_W2, proj_b2, lin_l_W2, lin_l_b2, lin_r_W2,
           post_W, post_b):
    f32 = jnp.float32
    pad_n = NP - N_NODES

    node_feat = jnp.pad(node_feat.astype(f32), ((0, pad_n), (0, 0)))
    node_config = jnp.pad(node_config_feat.astype(f32), ((0, pad_n), (0, 0)))
    opcode_col = jnp.pad(node_opcode.astype(jnp.int32),
                         (0, pad_n)).reshape(NP, 1)
    batch_pad = jnp.pad(batch.astype(jnp.int32), (0, pad_n))
    batch3d = batch_pad.reshape(NB, 1, BK)
    batchcol = batch_pad.reshape(NP, 1)

    srcp = jnp.pad(edge_index[0].astype(jnp.int32),
                   (0, EP - N_EDGES)).reshape(NCHT, CK)
    dstp = jnp.pad(edge_index[1].astype(jnp.int32), (0, EP - N_EDGES),
                   constant_values=N_NODES).reshape(NCHT, CK)

    zeros32 = jnp.zeros((RPS, 32), f32)
    zeros16 = jnp.zeros((RPS, 16), f32)
    ones16 = jnp.ones((CK, 16), f32)

    wa = lin_W[:140]
    wmid = lin_W[140:172]
    wc = lin_W[172:]
    lin_b2d = lin_b.reshape(1, 128)

    cnt = _sc_counts(dstp, zeros16, ones16)
    inv = _inv_counts(cnt)

    y0, y1, z = _input_layer0(node_feat, node_config, opcode_col, op_emb,
                              wa, wmid, wc, lin_b2d,
                              proj_W0, proj_b0.reshape(1, -1), lin_l_W0,
                              lin_r_W0, lin_l_b0.reshape(1, -1))
    agg = _sc_aggregate(y0, y1, srcp, dstp, zeros32)

    for pw, pb, lw, lb, rw in [
        (proj_W1, proj_b1.reshape(1, -1), lin_l_W1, lin_l_b1.reshape(1, -1),
         lin_r_W1),
        (proj_W2, proj_b2.reshape(1, -1), lin_l_W2, lin_l_b2.reshape(1, -1),
         lin_r_W2),
    ]:
        x = _combine(agg, inv, z)
        y0, y1, z = _layer_matmuls(x, pw, pb, lw, rw, lb)
        agg = _sc_aggregate(y0, y1, srcp, dstp, zeros32)

    return _pool_combine(agg, inv, z, batch3d, batchcol, post_W,
                         post_b.reshape(1, 1))


# speedup vs baseline: 1.3875x; 1.3875x over previous
import functools

import jax
import jax.numpy as jnp
from jax import lax
from jax.experimental import pallas as pl
from jax.experimental.pallas import tpu as pltpu
from jax.experimental.pallas import tpu_sc as plsc

N_NODES = 50000
N_EDGES = 800000
N_OPS = 120
H = 64
G = 512

BK = 512
NP = 50176
NB = NP // BK

NSC = 16
CK = 200
NCH = 250
CKC = 1000
NCH_C = 25
RPS = NP // NSC

_SC_MESH = dict(core_axis_name="c", subcore_axis_name="s")
_SC_PARAMS = pltpu.CompilerParams(use_tc_tiling_on_sc=False)


def _sc_aggregate(y0, y1, edge_index, zeros32):
    @functools.partial(
        pl.kernel,
        out_type=jax.ShapeDtypeStruct((2, NP, 32), jnp.float32),
        mesh=plsc.VectorSubcoreMesh(**_SC_MESH),
        scratch_types=[
            pltpu.VMEM((2, CK), jnp.int32),
            pltpu.VMEM((2, CK), jnp.int32),
            pltpu.VMEM((CK, 32), jnp.float32),
            pltpu.VMEM((CK, 32), jnp.float32),
            pltpu.VMEM_SHARED((NP, 32), jnp.float32),
            pltpu.SemaphoreType.DMA,
            pltpu.SemaphoreType.DMA,
        ],
        compiler_params=_SC_PARAMS,
    )
    def agg_kernel(y0_hbm, y1_hbm, ei_hbm, zero_hbm, out_hbm,
                   idx_a, idx_b, rows_a, rows_b, acc_sh, sem_a, sem_b):
        c = lax.axis_index("c")
        s = lax.axis_index("s")
        pltpu.sync_copy(zero_hbm, acc_sh.at[pl.ds(s * RPS, RPS)])
        plsc.subcore_barrier()
        base = s * NCH

        def load(g, idx):
            pltpu.sync_copy(
                ei_hbm.at[:, pl.ds((base + g) * CK, CK)], idx)

        def run(tbl):
            load(0, idx_a)
            pltpu.async_copy(tbl.at[idx_a.at[0]], rows_a, sem_a)

            def step(g, idx_x, rows_x, sem_x, idx_y, rows_y, sem_y, pre):
                load(g + 1, idx_y)
                pltpu.async_copy(tbl.at[idx_y.at[0]], rows_y, sem_y)
                pltpu.make_async_copy(tbl.at[idx_x.at[0]], rows_x,
                                      sem_x).wait()
                pltpu.sync_copy(rows_x, acc_sh.at[idx_x.at[1]], add=True)
                if pre:
                    load(g + 2, idx_x)
                    pltpu.async_copy(tbl.at[idx_x.at[0]], rows_x, sem_x)

            @pl.loop(0, NCH - 2, step=2)
            def _(g):
                step(g, idx_a, rows_a, sem_a, idx_b, rows_b, sem_b, True)
                pltpu.make_async_copy(tbl.at[idx_b.at[0]], rows_b,
                                      sem_b).wait()
                pltpu.sync_copy(rows_b, acc_sh.at[idx_b.at[1]], add=True)

            step(NCH - 2, idx_a, rows_a, sem_a, idx_b, rows_b, sem_b, False)
            pltpu.make_async_copy(tbl.at[idx_b.at[0]], rows_b, sem_b).wait()
            pltpu.sync_copy(rows_b, acc_sh.at[idx_b.at[1]], add=True)

        @pl.when(c == 0)
        def _():
            run(y0_hbm)

        @pl.when(c == 1)
        def _():
            run(y1_hbm)

        plsc.subcore_barrier()
        pltpu.sync_copy(acc_sh.at[pl.ds(s * RPS, RPS)],
                        out_hbm.at[c].at[pl.ds(s * RPS, RPS)])

    return agg_kernel(y0, y1, edge_index, zeros32)


def _sc_counts(edge_index, zeros16, ones16):
    @functools.partial(
        pl.kernel,
        out_type=jax.ShapeDtypeStruct((2, NP, 16), jnp.float32),
        mesh=plsc.VectorSubcoreMesh(**_SC_MESH),
        scratch_types=[
            pltpu.VMEM((CKC,), jnp.int32),
            pltpu.VMEM((CKC, 16), jnp.float32),
            pltpu.VMEM_SHARED((NP, 16), jnp.float32),
        ],
        compiler_params=_SC_PARAMS,
    )
    def cnt_kernel(ei_hbm, zero_hbm, ones_hbm, out_hbm,
                   idx_v, ones_v, acc_sh):
        c = lax.axis_index("c")
        s = lax.axis_index("s")
        pltpu.sync_copy(zero_hbm, acc_sh.at[pl.ds(s * RPS, RPS)])
        pltpu.sync_copy(ones_hbm, ones_v)
        plsc.subcore_barrier()
        base = (c * NSC + s) * NCH_C
        dst_hbm = ei_hbm.at[1]

        @pl.loop(0, NCH_C)
        def _(gi):
            pltpu.sync_copy(dst_hbm.at[pl.ds((base + gi) * CKC, CKC)], idx_v)
            pltpu.sync_copy(ones_v, acc_sh.at[idx_v], add=True)

        plsc.subcore_barrier()
        pltpu.sync_copy(acc_sh.at[pl.ds(s * RPS, RPS)],
                        out_hbm.at[c].at[pl.ds(s * RPS, RPS)])

    return cnt_kernel(edge_index, zeros16, ones16)


def _tot_counts(cnt):
    def body(c_r, o_r):
        cn = c_r[0, :, 0:1] + c_r[1, :, 0:1]
        o_r[...] = jnp.broadcast_to(jnp.maximum(cn, 1.0), (BK, 8))

    return pl.pallas_call(
        body,
        grid=(NB,),
        in_specs=[pl.BlockSpec((2, BK, 16), lambda i: (0, i, 0))],
        out_specs=pl.BlockSpec((BK, 8), lambda i: (i, 0)),
        out_shape=jax.ShapeDtypeStruct((NP, 8), jnp.float32),
    )(cnt)


def _input_layer0(node_feat, node_config, opcode3d, op_emb, wa, wmid, wc, b,
                  pw, pb, lw, rw, lb):
    def body(nf, cf, opc, emb, wa_r, wm_r, wc_r, b_r, pw_r, pb_r, lw_r,
             rw_r, lb_r, y0_r, y1_r, z_r):
        ot = jnp.dot(emb[...], wm_r[...], preferred_element_type=jnp.float32)
        op = opc[...]
        oh = (op == lax.broadcasted_iota(jnp.int32, (BK, N_OPS), 1)
              ).astype(jnp.float32)
        acc = jnp.dot(nf[...], wa_r[...], preferred_element_type=jnp.float32)
        acc += jnp.dot(oh, ot, preferred_element_type=jnp.float32)
        acc += jnp.dot(cf[...], wc_r[...], preferred_element_type=jnp.float32)
        acc += b_r[...]
        x = jnp.maximum(acc, 0.0)
        h = jnp.maximum(
            jnp.dot(x, pw_r[...], preferred_element_type=jnp.float32)
            + pb_r[...], 0.0)
        y = jnp.dot(h, lw_r[...], preferred_element_type=jnp.float32)
        y0_r[...] = y[:, :32]
        y1_r[...] = y[:, 32:]
        z_r[...] = (jnp.dot(h, rw_r[...], preferred_element_type=jnp.float32)
                    + lb_r[...])

    return pl.pallas_call(
        body,
        grid=(NB,),
        in_specs=[
            pl.BlockSpec((BK, 140), lambda i: (i, 0)),
            pl.BlockSpec((BK, 18), lambda i: (i, 0)),
            pl.BlockSpec((BK, 1), lambda i: (i, 0)),
            pl.BlockSpec((N_OPS, 32), lambda i: (0, 0)),
            pl.BlockSpec((140, 128), lambda i: (0, 0)),
            pl.BlockSpec((32, 128), lambda i: (0, 0)),
            pl.BlockSpec((18, 128), lambda i: (0, 0)),
            pl.BlockSpec((1, 128), lambda i: (0, 0)),
            pl.BlockSpec((128, 128), lambda i: (0, 0)),
            pl.BlockSpec((1, 128), lambda i: (0, 0)),
            pl.BlockSpec((128, H), lambda i: (0, 0)),
            pl.BlockSpec((128, H), lambda i: (0, 0)),
            pl.BlockSpec((1, H), lambda i: (0, 0)),
        ],
        out_specs=[
            pl.BlockSpec((BK, 32), lambda i: (i, 0)),
            pl.BlockSpec((BK, 32), lambda i: (i, 0)),
            pl.BlockSpec((BK, H), lambda i: (i, 0)),
        ],
        out_shape=[
            jax.ShapeDtypeStruct((NP, 32), jnp.float32),
            jax.ShapeDtypeStruct((NP, 32), jnp.float32),
            jax.ShapeDtypeStruct((NP, H), jnp.float32),
        ],
    )(node_feat, node_config, opcode3d, op_emb, wa, wmid, wc, b,
      pw, pb, lw, rw, lb)


def _combine(agg, cn, z):
    def body(a_r, c_r, z_r, o_r):
        s = jnp.concatenate([a_r[0], a_r[1]], axis=1)
        x = s / c_r[:, 0:1] + z_r[...]
        nrm = jnp.sqrt(jnp.sum(x * x, axis=1, keepdims=True))
        o_r[...] = x / jnp.maximum(nrm, 1e-12)

    return pl.pallas_call(
        body,
        grid=(NB,),
        in_specs=[
            pl.BlockSpec((2, BK, 32), lambda i: (0, i, 0)),
            pl.BlockSpec((BK, 8), lambda i: (i, 0)),
            pl.BlockSpec((BK, H), lambda i: (i, 0)),
        ],
        out_specs=pl.BlockSpec((BK, H), lambda i: (i, 0)),
        out_shape=jax.ShapeDtypeStruct((NP, H), jnp.float32),
    )(agg, cn, z)


def _layer_matmuls(x, pw, pb, lw, rw, lb):
    d = x.shape[1]

    def body(x_r, pw_r, pb_r, lw_r, rw_r, lb_r, y0_r, y1_r, z_r):
        h = jnp.maximum(
            jnp.dot(x_r[...], pw_r[...], preferred_element_type=jnp.float32)
            + pb_r[...], 0.0)
        y = jnp.dot(h, lw_r[...], preferred_element_type=jnp.float32)
        y0_r[...] = y[:, :32]
        y1_r[...] = y[:, 32:]
        z_r[...] = (jnp.dot(h, rw_r[...], preferred_element_type=jnp.float32)
                    + lb_r[...])

    return pl.pallas_call(
        body,
        grid=(NB,),
        in_specs=[
            pl.BlockSpec((BK, d), lambda i: (i, 0)),
            pl.BlockSpec((d, d), lambda i: (0, 0)),
            pl.BlockSpec((1, d), lambda i: (0, 0)),
            pl.BlockSpec((d, H), lambda i: (0, 0)),
            pl.BlockSpec((d, H), lambda i: (0, 0)),
            pl.BlockSpec((1, H), lambda i: (0, 0)),
        ],
        out_specs=[
            pl.BlockSpec((BK, 32), lambda i: (i, 0)),
            pl.BlockSpec((BK, 32), lambda i: (i, 0)),
            pl.BlockSpec((BK, H), lambda i: (i, 0)),
        ],
        out_shape=[
            jax.ShapeDtypeStruct((NP, 32), jnp.float32),
            jax.ShapeDtypeStruct((NP, 32), jnp.float32),
            jax.ShapeDtypeStruct((NP, H), jnp.float32),
        ],
    )(x, pw, pb, lw, rw, lb)


def _pool_combine(agg, cn, z, batch3d, batchcol, post_w, post_b):
    def body(a_r, c_r, z_r, b_r, bc_r, pw_r, pb_r, o_r, smax, ssum, scnt):
        i = pl.program_id(0)

        @pl.when(i == 0)
        def _():
            smax[...] = jnp.full((G, H), -jnp.inf, jnp.float32)
            ssum[...] = jnp.zeros((G, H), jnp.float32)
            scnt[...] = jnp.zeros((G, 8), jnp.float32)

        s = jnp.concatenate([a_r[0], a_r[1]], axis=1)
        x = s / c_r[:, 0:1] + z_r[...]
        nrm = jnp.sqrt(jnp.sum(x * x, axis=1, keepdims=True))
        x = x / jnp.maximum(nrm, 1e-12)

        nrow = i * BK + lax.broadcasted_iota(jnp.int32, (1, BK), 1)
        bm = jnp.where(nrow < N_NODES, b_r[0], -1)
        onehot = (bm == lax.broadcasted_iota(jnp.int32, (G, BK), 0)
                  ).astype(jnp.float32)
        ssum[...] += jnp.dot(onehot, x, preferred_element_type=jnp.float32)
        scnt[...] += jnp.dot(onehot, jnp.ones((BK, 8), jnp.float32),
                             preferred_element_type=jnp.float32)

        g_lo = jnp.min(jnp.where(nrow < N_NODES, bm, G))
        g_hi = jnp.max(bm)

        ncol = i * BK + lax.broadcasted_iota(jnp.int32, (BK, 1), 0)
        bmc = jnp.where(ncol < N_NODES, bc_r[...], -1)

        def upd(g, carry):
            col = jnp.max(jnp.where(bmc == g, x, -jnp.inf),
                          axis=0, keepdims=True)
            smax[pl.ds(g, 1), :] = jnp.maximum(smax[pl.ds(g, 1), :], col)
            return carry

        lax.fori_loop(g_lo, g_hi + 1, upd, 0)

        @pl.when(i == NB - 1)
        def _():
            mean = ssum[...] / jnp.maximum(scnt[...][:, 0:1], 1.0)
            mx = smax[...]
            mx = jnp.where(mx == -jnp.inf, 0.0, mx)
            gv = mx + mean
            gv = gv / jnp.sqrt(jnp.sum(gv * gv, axis=1, keepdims=True))
            o_r[...] = (jnp.dot(gv, pw_r[...],
                                preferred_element_type=jnp.float32)
                        + pb_r[...])

    return pl.pallas_call(
        body,
        grid=(NB,),
        in_specs=[
            pl.BlockSpec((2, BK, 32), lambda i: (0, i, 0)),
            pl.BlockSpec((BK, 8), lambda i: (i, 0)),
            pl.BlockSpec((BK, H), lambda i: (i, 0)),
            pl.BlockSpec((1, 1, BK), lambda i: (i, 0, 0)),
            pl.BlockSpec((BK, 1), lambda i: (i, 0)),
            pl.BlockSpec((H, 1), lambda i: (0, 0)),
            pl.BlockSpec((1, 1), lambda i: (0, 0)),
        ],
        out_specs=pl.BlockSpec((G, 1), lambda i: (0, 0)),
        out_shape=jax.ShapeDtypeStruct((G, 1), jnp.float32),
        scratch_shapes=[
            pltpu.VMEM((G, H), jnp.float32),
            pltpu.VMEM((G, H), jnp.float32),
            pltpu.VMEM((G, 8), jnp.float32),
        ],
    )(agg, cn, z, batch3d, batchcol, post_w, post_b)


def kernel(node_feat, node_config_feat, node_opcode, edge_index, batch, op_emb,
           lin_W, lin_b,
           proj_W0, proj_b0, lin_l_W0, lin_l_b0, lin_r_W0,
           proj_W1, proj_b1, lin_l_W1, lin_l_b1, lin_r_W1,
           proj_W2, proj_b2, lin_l_W2, lin_l_b2, lin_r_W2,
           post_W, post_b):
    f32 = jnp.float32
    pad_n = NP - N_NODES

    node_feat = jnp.pad(node_feat.astype(f32), ((0, pad_n), (0, 0)))
    node_config = jnp.pad(node_config_feat.astype(f32), ((0, pad_n), (0, 0)))
    opcode_col = jnp.pad(node_opcode.astype(jnp.int32),
                         (0, pad_n)).reshape(NP, 1)
    batch_pad = jnp.pad(batch.astype(jnp.int32), (0, pad_n))
    batch3d = batch_pad.reshape(NB, 1, BK)
    batchcol = batch_pad.reshape(NP, 1)

    edge_index = edge_index.astype(jnp.int32)

    zeros32 = jnp.zeros((RPS, 32), f32)
    zeros16 = jnp.zeros((RPS, 16), f32)
    ones16 = jnp.ones((CKC, 16), f32)

    wa = lin_W[:140]
    wmid = lin_W[140:172]
    wc = lin_W[172:]
    lin_b2d = lin_b.reshape(1, 128)

    cnt = _sc_counts(edge_index, zeros16, ones16)
    cn = _tot_counts(cnt)

    y0, y1, z = _input_layer0(node_feat, node_config, opcode_col, op_emb,
                              wa, wmid, wc, lin_b2d,
                              proj_W0, proj_b0.reshape(1, -1), lin_l_W0,
                              lin_r_W0, lin_l_b0.reshape(1, -1))
    agg = _sc_aggregate(y0, y1, edge_index, zeros32)

    for pw, pb, lw, lb, rw in [
        (proj_W1, proj_b1.reshape(1, -1), lin_l_W1, lin_l_b1.reshape(1, -1),
         lin_r_W1),
        (proj_W2, proj_b2.reshape(1, -1), lin_l_W2, lin_l_b2.reshape(1, -1),
         lin_r_W2),
    ]:
        x = _combine(agg, cn, z)
        y0, y1, z = _layer_matmuls(x, pw, pb, lw, rw, lb)
        agg = _sc_aggregate(y0, y1, edge_index, zeros32)

    return _pool_combine(agg, cn, z, batch3d, batchcol, post_W,
                         post_b.reshape(1, 1))
